# baseline (speedup 1.0000x reference)
import functools

import jax
import jax.numpy as jnp
from jax import lax
from jax.experimental import pallas as pl
from jax.experimental.pallas import tpu as pltpu
from jax.experimental.pallas import tpu_sc as plsc

_K = 1024
_D = 256
_TOK_BLK = 512

_NW = 32
_NTOK = 4608
_BPW = _NTOK // _NW


def _dist_argmin_body(xsq_ref, esq_ref, x_ref, emb_ref, idx_ref, cnt_ref,
                      usage_ref):
    i = pl.program_id(0)
    t = x_ref.shape[0]
    cross = lax.dot_general(
        x_ref[...], emb_ref[...], (((1,), (1,)), ((), ())),
        preferred_element_type=jnp.float32)
    xsq_col = xsq_ref[...].reshape(t, 1)
    esq_row = esq_ref[...].reshape(1, _K)
    s = (xsq_col + esq_row) - 2.0 * cross
    dist = jnp.sqrt(jnp.maximum(s, 0.0))
    m = jnp.min(dist, axis=-1, keepdims=True)
    iota = lax.broadcasted_iota(jnp.int32, (t, _K), 1)
    idx = jnp.min(jnp.where(dist == m, iota, _K), axis=-1).astype(jnp.int32)
    idx_ref[...] = idx.reshape(1, 1, t)

    onehot = (idx[:, None] == iota).astype(jnp.float32)
    blk_cnt = jnp.sum(onehot, axis=0).reshape(1, _K)

    @pl.when(i == 0)
    def _init():
        cnt_ref[...] = jnp.zeros_like(cnt_ref)

    cnt_ref[...] += blk_cnt

    @pl.when(i == pl.num_programs(0) - 1)
    def _finish():
        zero_cnt = jnp.sum((cnt_ref[...] == 0.0).astype(jnp.float32))
        usage_ref[...] = jnp.full((1, 1), zero_cnt * (1.0 / _K), jnp.float32)


def _dist_argmin(xsq3, esq2, x2, embedding, interpret=False):
    grid = _NTOK // _TOK_BLK
    return pl.pallas_call(
        _dist_argmin_body,
        grid=(grid,),
        in_specs=[
            pl.BlockSpec((1, 1, _TOK_BLK), lambda i: (i, 0, 0)),
            pl.BlockSpec((1, _K), lambda i: (0, 0)),
            pl.BlockSpec((_TOK_BLK, _D), lambda i: (i, 0)),
            pl.BlockSpec((_K, _D), lambda i: (0, 0)),
        ],
        out_specs=[
            pl.BlockSpec((1, 1, _TOK_BLK), lambda i: (i, 0, 0)),
            pl.BlockSpec((1, _K), lambda i: (0, 0)),
            pl.BlockSpec((1, 1), lambda i: (0, 0)),
        ],
        out_shape=[
            jax.ShapeDtypeStruct((grid, 1, _TOK_BLK), jnp.int32),
            jax.ShapeDtypeStruct((1, _K), jnp.float32),
            jax.ShapeDtypeStruct((1, 1), jnp.float32),
        ],
        interpret=interpret,
    )(xsq3, esq2, x2, embedding)


def _sc_gather(embedding, idx_flat):
    mesh = plsc.VectorSubcoreMesh(core_axis_name="c", subcore_axis_name="s")

    @functools.partial(
        pl.kernel,
        mesh=mesh,
        out_type=jax.ShapeDtypeStruct((_NTOK, _D), jnp.float32),
        scratch_types=[
            pltpu.VMEM((_BPW,), jnp.int32),
            pltpu.VMEM((_BPW, _D), jnp.float32),
            pltpu.SemaphoreType.DMA,
        ],
    )
    def k(emb_hbm, idx_hbm, out_hbm, idx_v, rows_v, sem):
        wid = lax.axis_index("s") * 2 + lax.axis_index("c")
        base = wid * _BPW
        pltpu.sync_copy(idx_hbm.at[pl.ds(base, _BPW)], idx_v)
        pltpu.async_copy(emb_hbm.at[idx_v], rows_v, sem).wait()
        pltpu.sync_copy(rows_v, out_hbm.at[pl.ds(base, _BPW)])

    return k(embedding, idx_flat)


def kernel(x, embedding):
    x = x.astype(jnp.float32)
    b, n, _ = x.shape
    x_sq = jnp.sum(x * x, axis=-1)
    e_sq = jnp.sum(embedding * embedding, axis=-1)
    x2 = x.reshape(_NTOK, _D)
    xsq3 = x_sq.reshape(_NTOK // _TOK_BLK, 1, _TOK_BLK)
    esq2 = e_sq.reshape(1, _K)
    idx3, _cnt, usage = _dist_argmin(xsq3, esq2, x2, embedding)
    idx_flat = idx3.reshape(_NTOK)
    quantized = _sc_gather(embedding, idx_flat).reshape(b, n, _D)
    embed_idx = idx_flat.reshape(b, n)
    code_usage = usage.reshape(())
    return (quantized, embed_idx, code_usage)
